# baseline (speedup 1.0000x reference)
import numpy as np
import jax
import jax.numpy as jnp
from jax import lax
from jax.experimental import pallas as pl
from jax.experimental.pallas import tpu as pltpu
from jax.experimental.pallas import tpu_sc as plsc

_VOCAB = 100000
_DEPTH = 64
_BATCH = 4096
_SEQ = 200


def _positional_table():
    effective_depth = _DEPTH / 2
    depth_vector = np.repeat(np.arange(effective_depth), 2)
    frequency_vector = 1 / 10000 ** (2 * depth_vector / _DEPTH)
    sequence_vector = np.arange(_SEQ)
    pos = sequence_vector.reshape([-1, 1]) * frequency_vector.reshape([1, -1])
    pos[:, ::2] = np.sin(pos[:, ::2])
    pos[:, 1::2] = np.cos(pos[:, 1::2])
    return pos.astype(np.float32)


_NC = 2
_NS = 16
_NW = _NC * _NS

_SPLITS = ((0, 128), (128, 72))
_NBUF = 4
_LOOK = 2
_K = 2
_HB = _BATCH // _K


def _sc_body(w_hbm, x_hbm, pos_hbm, out_hbm, idx_v, pos_v, rows_v,
             gsem, ssem):
    spw = idx_v.shape[0]
    wid = lax.axis_index("s") * _NC + lax.axis_index("c")
    sbase = wid * spw

    pltpu.sync_copy(pos_hbm, pos_v)
    pltpu.sync_copy(x_hbm.at[pl.ds(sbase, spw)], idx_v)

    scale = jnp.float32(np.sqrt(float(_DEPTH)))

    def gather_part(t, s, off, n):
        return pltpu.make_async_copy(
            w_hbm.at[idx_v.at[t, pl.ds(off, n)]],
            rows_v.at[s, pl.ds(off, n)],
            gsem[s])

    def start_gather(t, s):
        for off, n in _SPLITS:
            gather_part(t, s, off, n).start()

    def wait_gather(t, s):
        for off, n in _SPLITS:
            gather_part(t, s, off, n).wait()

    def scatter(t, s):
        return pltpu.make_async_copy(rows_v.at[s], out_hbm.at[sbase + t],
                                     ssem[s])

    for s in range(_LOOK):
        start_gather(s, s)

    def outer(o, _):
        for b in range(_NBUF):
            t = o * _NBUF + b
            sp = (b + _LOOK) % _NBUF

            @pl.when(t + _LOOK < spw)
            def _():
                @pl.when(t >= _NBUF - _LOOK)
                def _():
                    scatter(t + _LOOK - _NBUF, sp).wait()
                start_gather(t + _LOOK, sp)

            wait_gather(t, b)

            @plsc.parallel_loop(0, _SEQ, unroll=4)
            def _(e):
                for p in range(_DEPTH // 16):
                    sl = pl.ds(p * 16, 16)
                    rows_v[b, e, sl] = rows_v[b, e, sl] * scale + pos_v[e, sl]

            scatter(t, b).start()
        return 0

    lax.fori_loop(0, spw // _NBUF, outer, 0)

    for s in range(_NBUF):
        scatter(spw - _NBUF + s, s).wait()


def _sc_gather(W, xk, pos, mesh):
    spw = xk.shape[0] // _NW
    return pl.kernel(
        _sc_body,
        mesh=mesh,
        compiler_params=pltpu.CompilerParams(use_tc_tiling_on_sc=False),
        out_type=jax.ShapeDtypeStruct((xk.shape[0], _SEQ, _DEPTH),
                                      jnp.float32),
        scratch_types=[
            pltpu.VMEM((spw, _SEQ), jnp.int32),
            pltpu.VMEM((_SEQ, _DEPTH), jnp.float32),
            pltpu.VMEM((_NBUF, _SEQ, _DEPTH), jnp.float32),
            [pltpu.SemaphoreType.DMA] * _NBUF,
            [pltpu.SemaphoreType.DMA] * _NBUF,
        ],
    )(W, xk, pos)


_RPB = 128 * _SEQ * _DEPTH // 128
_BPH = _HB // 128


def _tc_transpose(y_ref, out_ref):
    x = y_ref[...]
    t = jnp.swapaxes(x.reshape(128, _RPB), 0, 1)
    out_ref[...] = t.reshape(_SEQ, 8, 1, 8, 128)


def _tc_transpose_alias(y_ref, acc_ref, out_ref):
    del acc_ref
    _tc_transpose(y_ref, out_ref)


_OUT5 = jax.ShapeDtypeStruct((_SEQ, 8, _NW, 8, 128), jnp.float32)


@jax.jit
def _embed(x, W):
    pos = jnp.asarray(_positional_table())
    mesh = plsc.VectorSubcoreMesh(core_axis_name="c", subcore_axis_name="s")

    ys = [_sc_gather(W, x[k * _HB:(k + 1) * _HB], pos, mesh)
          for k in range(_K)]
    y2s = [y.reshape(_HB * _SEQ * _DEPTH // 128, 128) for y in ys]

    out5 = pl.pallas_call(
        _tc_transpose,
        grid=(_BPH,),
        in_specs=[pl.BlockSpec((_RPB, 128), lambda i: (i, 0))],
        out_specs=pl.BlockSpec((_SEQ, 8, 1, 8, 128),
                               lambda i: (0, 0, i, 0, 0)),
        out_shape=_OUT5,
    )(y2s[0])
    for k in range(1, _K):
        out5 = pl.pallas_call(
            _tc_transpose_alias,
            grid=(_BPH,),
            in_specs=[
                pl.BlockSpec((_RPB, 128), lambda i: (i, 0)),
                pl.BlockSpec(memory_space=pl.ANY),
            ],
            out_specs=pl.BlockSpec((_SEQ, 8, 1, 8, 128),
                                   lambda i, k=k: (0, 0, k * _BPH + i, 0, 0)),
            out_shape=_OUT5,
            input_output_aliases={1: 0},
        )(y2s[k], out5)
    return out5.transpose(2, 4, 0, 1, 3).reshape(_BATCH, _SEQ, _DEPTH)


def kernel(x, W):
    return _embed(x, W)


# speedup vs baseline: 1.0140x; 1.0140x over previous
import numpy as np
import jax
import jax.numpy as jnp
from jax import lax
from jax.experimental import pallas as pl
from jax.experimental.pallas import tpu as pltpu
from jax.experimental.pallas import tpu_sc as plsc

_VOCAB = 100000
_DEPTH = 64
_BATCH = 4096
_SEQ = 200


def _positional_table():
    effective_depth = _DEPTH / 2
    depth_vector = np.repeat(np.arange(effective_depth), 2)
    frequency_vector = 1 / 10000 ** (2 * depth_vector / _DEPTH)
    sequence_vector = np.arange(_SEQ)
    pos = sequence_vector.reshape([-1, 1]) * frequency_vector.reshape([1, -1])
    pos[:, ::2] = np.sin(pos[:, ::2])
    pos[:, 1::2] = np.cos(pos[:, 1::2])
    return pos.astype(np.float32)


_NC = 2
_NS = 16
_NW = _NC * _NS

_SPW = _BATCH // _NW
_SPLITS = ((0, 128), (128, 72))
_NBUF = 4
_LOOK = 2


def _sc_body(w_hbm, x_hbm, pos_hbm, out_hbm, idx_v, pos_v, rows_v,
             gsem, ssem):
    wid = lax.axis_index("s") * _NC + lax.axis_index("c")
    sbase = wid * _SPW

    pltpu.sync_copy(pos_hbm, pos_v)
    pltpu.sync_copy(x_hbm.at[pl.ds(sbase, _SPW)], idx_v)

    scale = jnp.float32(np.sqrt(float(_DEPTH)))

    def gather_part(t, s, off, n):
        return pltpu.make_async_copy(
            w_hbm.at[idx_v.at[t, pl.ds(off, n)]],
            rows_v.at[s, pl.ds(off, n)],
            gsem[s])

    def start_gather(t, s):
        for off, n in _SPLITS:
            gather_part(t, s, off, n).start()

    def wait_gather(t, s):
        for off, n in _SPLITS:
            gather_part(t, s, off, n).wait()

    def scatter(t, s):
        return pltpu.make_async_copy(rows_v.at[s], out_hbm.at[sbase + t],
                                     ssem[s])

    for s in range(_LOOK):
        start_gather(s, s)

    def outer(o, _):
        for b in range(_NBUF):
            t = o * _NBUF + b
            sp = (b + _LOOK) % _NBUF

            @pl.when(t + _LOOK < _SPW)
            def _():
                @pl.when(t >= _NBUF - _LOOK)
                def _():
                    scatter(t + _LOOK - _NBUF, sp).wait()
                start_gather(t + _LOOK, sp)

            wait_gather(t, b)

            @plsc.parallel_loop(0, _SEQ, unroll=4)
            def _(e):
                for p in range(_DEPTH // 16):
                    sl = pl.ds(p * 16, 16)
                    rows_v[b, e, sl] = rows_v[b, e, sl] * scale + pos_v[e, sl]

            scatter(t, b).start()
        return 0

    lax.fori_loop(0, _SPW // _NBUF, outer, 0)

    for s in range(_NBUF):
        scatter(_SPW - _NBUF + s, s).wait()


_RPB = _SPW * _SEQ * _DEPTH // 128


def _tc_transpose(y_ref, out_ref):
    x = y_ref[...]
    t = jnp.swapaxes(x.reshape(128, _RPB), 0, 1)
    out_ref[...] = t.reshape(_SEQ, 8, 1, 8, 128)


@jax.jit
def _embed(x, W):
    pos = jnp.asarray(_positional_table())
    mesh = plsc.VectorSubcoreMesh(core_axis_name="c", subcore_axis_name="s")
    y = pl.kernel(
        _sc_body,
        mesh=mesh,
        compiler_params=pltpu.CompilerParams(use_tc_tiling_on_sc=False),
        out_type=jax.ShapeDtypeStruct((_BATCH, _SEQ, _DEPTH), jnp.float32),
        scratch_types=[
            pltpu.VMEM((_SPW, _SEQ), jnp.int32),
            pltpu.VMEM((_SEQ, _DEPTH), jnp.float32),
            pltpu.VMEM((_NBUF, _SEQ, _DEPTH), jnp.float32),
            [pltpu.SemaphoreType.DMA] * _NBUF,
            [pltpu.SemaphoreType.DMA] * _NBUF,
        ],
    )(W, x, pos)
    y2 = y.reshape(_BATCH * _SEQ * _DEPTH // 128, 128)
    out5 = pl.pallas_call(
        _tc_transpose,
        grid=(_NW,),
        in_specs=[pl.BlockSpec((_RPB, 128), lambda i: (i, 0))],
        out_specs=pl.BlockSpec((_SEQ, 8, 1, 8, 128),
                               lambda i: (0, 0, i, 0, 0)),
        out_shape=jax.ShapeDtypeStruct((_SEQ, 8, _NW, 8, 128), jnp.float32),
    )(y2)
    return out5.transpose(2, 4, 0, 1, 3).reshape(_BATCH, _SEQ, _DEPTH)


def kernel(x, W):
    return _embed(x, W)
